# baseline (speedup 1.0000x reference)
import jax
import jax.numpy as jnp
from jax import lax
from jax.experimental import pallas as pl
from jax.experimental.pallas import tpu as pltpu

_BM = 1024


def _get_score_kernel(n, nb, x_ref, w_ref, xout_ref, score_ref, xs_ref, acc_ref):
    i = pl.program_id(0)

    @pl.when(i == 0)
    def _init():
        acc_ref[0] = 0.0

    @pl.when(i < nb)
    def _pass1():
        xv = x_ref[...]
        w = w_ref[...]
        xs_ref[pl.ds(i * _BM, _BM), :] = xv
        s_row = lax.dot_general(
            w, xv, (((1,), (1,)), ((), ())), preferred_element_type=jnp.float32
        )
        col = lax.broadcasted_iota(jnp.int32, (1, _BM), 1) + i * _BM
        s_row = jnp.where(col < n, s_row, 0.0)
        acc_ref[0] += jnp.sum(s_row)

    @pl.when(i >= nb)
    def _pass2():
        j = i - nb
        xv = xs_ref[pl.ds(j * _BM, _BM), :]
        w = w_ref[...]
        m = acc_ref[0] / n
        inv_norm = lax.rsqrt(jnp.sum(w * w))
        s_col = lax.dot_general(
            xv, w, (((1,), (1,)), ((), ())), preferred_element_type=jnp.float32
        )
        s_row = lax.dot_general(
            w, xv, (((1,), (1,)), ((), ())), preferred_element_type=jnp.float32
        )
        xout_ref[...] = xv * jnp.tanh((s_col - m) * inv_norm)
        score_ref[...] = jnp.tanh((s_row - m) * inv_norm)


def kernel(x, edge_index, weight):
    n, d = x.shape
    nb = (n + _BM - 1) // _BM
    n_pad = nb * _BM

    def body(x_ref, w_ref, xout_ref, score_ref, xs_ref, acc_ref):
        _get_score_kernel(n, nb, x_ref, w_ref, xout_ref, score_ref, xs_ref, acc_ref)

    x_out, score = pl.pallas_call(
        body,
        grid=(2 * nb,),
        in_specs=[
            pl.BlockSpec((_BM, d), lambda i: (jnp.minimum(i, nb - 1), 0)),
            pl.BlockSpec((1, d), lambda i: (0, 0)),
        ],
        out_specs=[
            pl.BlockSpec((_BM, d), lambda i: (jnp.maximum(i - nb, 0), 0)),
            pl.BlockSpec((1, _BM), lambda i: (0, jnp.maximum(i - nb, 0))),
        ],
        out_shape=(
            jax.ShapeDtypeStruct((n, d), x.dtype),
            jax.ShapeDtypeStruct((1, n), x.dtype),
        ),
        scratch_shapes=[
            pltpu.VMEM((n_pad, d), jnp.float32),
            pltpu.SMEM((1,), jnp.float32),
        ],
    )(x, weight)
    return x_out, score


# speedup vs baseline: 1.7394x; 1.7394x over previous
import jax
import jax.numpy as jnp
from jax import lax
from jax.experimental import pallas as pl


def _get_score_kernel(x_ref, w_ref, xout_ref, score_ref):
    xv = x_ref[...]
    w = w_ref[...]
    s_col = lax.dot_general(
        xv, w, (((1,), (1,)), ((), ())), preferred_element_type=jnp.float32
    )
    m = jnp.mean(s_col)
    inv_norm = lax.rsqrt(jnp.sum(w * w))
    sc_col = jnp.tanh((s_col - m) * inv_norm)
    xout_ref[...] = xv * sc_col
    score_ref[...] = lax.transpose(sc_col, (1, 0))


def kernel(x, edge_index, weight):
    n, d = x.shape
    x_out, score = pl.pallas_call(
        _get_score_kernel,
        out_shape=(
            jax.ShapeDtypeStruct((n, d), x.dtype),
            jax.ShapeDtypeStruct((1, n), x.dtype),
        ),
    )(x, weight)
    return x_out, score
